# baseline (speedup 1.0000x reference)
import functools

import jax
import jax.numpy as jnp
from jax import lax
from jax.experimental import pallas as pl
from jax.experimental.pallas import tpu as pltpu
from jax.experimental.pallas import tpu_sc as plsc

NC = 2
NS = 16
NW = NC * NS
NPAD = 10112
EBLK = 512
HR = 80


def _sc_segment_sum(nblk, feat, with_cnt):
    rows_per_sub = NPAD // NS
    kmax2 = nblk // (2 * NW)
    mesh = plsc.VectorSubcoreMesh(core_axis_name="c", subcore_axis_name="s")

    out_type = [jax.ShapeDtypeStruct((NC * NPAD, feat), jnp.float32)]
    scratch = [
        pltpu.VMEM((4, 128), jnp.int32),
        pltpu.VMEM((4, 128), jnp.int32),
        pltpu.VMEM((4, 128), jnp.int32),
        pltpu.VMEM((4, 128), jnp.int32),
        pltpu.VMEM((128, feat), jnp.float32),
        pltpu.VMEM((128, feat), jnp.float32),
        pltpu.VMEM_SHARED((NPAD, feat), jnp.float32),
        pltpu.SemaphoreType.DMA,
        pltpu.SemaphoreType.DMA,
    ]
    if with_cnt:
        out_type.append(jax.ShapeDtypeStruct((NC * HR, 128), jnp.float32))
        scratch += [
            pltpu.VMEM((HR, 128), jnp.float32),
            pltpu.VMEM((HR,), jnp.int32),
            pltpu.VMEM_SHARED((HR, 128), jnp.float32),
        ]

    @functools.partial(pl.kernel, mesh=mesh, out_type=out_type,
                       scratch_types=scratch)
    def f(x_hbm, src_hbm, dst_hbm, zf_hbm, iota_hbm, *rest):
        if with_cnt:
            (agg_out, cnt_out, srcA, dstA, srcB, dstB, rows0, rows1, agg_s,
             sem0, sem1, hist, iota_v, cnt_s) = rest
        else:
            (agg_out, srcA, dstA, srcB, dstB, rows0, rows1, agg_s,
             sem0, sem1) = rest
        rowsb = (rows0, rows1)
        semb = (sem0, sem1)
        srcb = (srcA, srcB)
        dstb = (dstA, dstB)
        c = lax.axis_index("c")
        s = lax.axis_index("s")
        wid = s * NC + c
        base = s * rows_per_sub

        pltpu.sync_copy(zf_hbm, agg_s.at[pl.ds(base, rows_per_sub)])
        if with_cnt:
            pltpu.sync_copy(zf_hbm.at[pl.ds(0, HR)], hist)
            pltpu.sync_copy(iota_hbm, iota_v)
            @pl.when(s == 0)
            def _():
                pltpu.sync_copy(zf_hbm.at[pl.ds(0, HR)], cnt_s)
        plsc.subcore_barrier()

        iot = lax.iota(jnp.int32, 16)

        def histo(dref, j):
            def hstep(t, carry):
                dv = dref[j, pl.ds(t * 16, 16)]
                for ll in range(16):
                    d = dv[ll]
                    hi = d // 128
                    lo = d % 128
                    off = (lo // 16) * 16
                    l16 = lo % 16
                    row = hist[hi, pl.ds(off, 16)]
                    row = row + jnp.where(iot == l16, 1.0, 0.0).astype(
                        jnp.float32)
                    hist[hi, pl.ds(off, 16)] = row
                return carry
            lax.fori_loop(0, 8, hstep, 0)

        def fire(sref, j, sl):
            pltpu.async_copy(x_hbm.at[sref.at[j]], rowsb[sl], semb[sl])

        def drain(sl):
            pltpu.make_async_copy(x_hbm.at[srcA.at[0]], rowsb[sl],
                                  semb[sl]).wait()

        b0 = wid
        pltpu.sync_copy(src_hbm.at[b0], srcA)
        pltpu.sync_copy(dst_hbm.at[b0], dstA)
        fire(srcA, 0, 0)
        fire(srcA, 1, 1)

        def pair(p, carry):
            bA = wid + (2 * p) * NW
            bB = bA + NW
            bA2 = bA + 2 * NW
            pltpu.sync_copy(src_hbm.at[bB], srcB)
            pltpu.sync_copy(dst_hbm.at[bB], dstB)
            for sidx in range(8):
                blk, j = divmod(sidx, 4)
                sl = sidx % 2
                if with_cnt:
                    histo(dstb[blk], j)
                drain(sl)
                pltpu.sync_copy(rowsb[sl], agg_s.at[dstb[blk].at[j]],
                                add=True)
                nidx = sidx + 2
                if nidx < 8:
                    fire(srcb[nidx // 4], nidx % 4, sl)
                elif nidx == 8:
                    pltpu.sync_copy(src_hbm.at[bA2], srcA)
                    pltpu.sync_copy(dst_hbm.at[bA2], dstA)
                    fire(srcA, 0, sl)
                else:
                    fire(srcA, 1, sl)
            return carry
        lax.fori_loop(0, kmax2, pair, 0, unroll=False)
        drain(0)
        drain(1)
        if with_cnt:
            pltpu.sync_copy(hist, cnt_s.at[iota_v], add=True)
        plsc.subcore_barrier()

        out_base = c * NPAD + base
        pltpu.sync_copy(agg_s.at[pl.ds(base, rows_per_sub)],
                        agg_out.at[pl.ds(out_base, rows_per_sub)])
        if with_cnt:
            @pl.when(s == 0)
            def _():
                pltpu.sync_copy(cnt_s, cnt_out.at[pl.ds(c * HR, HR)])

    return f


def _make_combine(n_nodes, feat, relu):
    blk = 2000
    g = n_nodes // blk

    def body(a0, a1, c0, c1, xb, wl, bb, wr, ob):
        agg = a0[...] + a1[...]
        cnt = c0[...] + c1[...]
        mean = agg / jnp.maximum(cnt, 1.0)
        acc = lax.dot_general(mean, wl[...], (((1,), (1,)), ((), ())),
                              preferred_element_type=jnp.float32)
        acc = acc + bb[...]
        acc = acc + lax.dot_general(xb[...], wr[...], (((1,), (1,)), ((), ())),
                                    preferred_element_type=jnp.float32)
        if relu:
            acc = jnp.maximum(acc, 0.0)
        ob[...] = acc

    return pl.pallas_call(
        body,
        grid=(g,),
        in_specs=[
            pl.BlockSpec((blk, feat), lambda i: (i, 0)),
            pl.BlockSpec((blk, feat), lambda i: (i, 0)),
            pl.BlockSpec((blk, 1), lambda i: (i, 0)),
            pl.BlockSpec((blk, 1), lambda i: (i, 0)),
            pl.BlockSpec((blk, feat), lambda i: (i, 0)),
            pl.BlockSpec((feat, feat), lambda i: (0, 0)),
            pl.BlockSpec((1, feat), lambda i: (0, 0)),
            pl.BlockSpec((feat, feat), lambda i: (0, 0)),
        ],
        out_specs=pl.BlockSpec((blk, feat), lambda i: (i, 0)),
        out_shape=jax.ShapeDtypeStruct((n_nodes, feat), jnp.float32),
    )


def kernel(x, edge_index, W1l, b1, W1r, W2l, b2, W2r):
    n, d = x.shape
    e = edge_index.shape[1]
    epad = -e % (2 * NW * EBLK)
    ghost = NW * EBLK
    src = jnp.concatenate([edge_index[0],
                           jnp.zeros((epad + ghost,), jnp.int32)])
    dst = jnp.concatenate([edge_index[1],
                           jnp.full((epad + ghost,), NPAD - 1, jnp.int32)])
    nblk = (e + epad) // EBLK
    src3d = src.reshape(nblk + NW, 4, 128)
    dst3d = dst.reshape(nblk + NW, 4, 128)
    rows_per_sub = NPAD // NS
    zf = jnp.zeros((rows_per_sub, d), jnp.float32)
    iota80 = jnp.arange(HR, dtype=jnp.int32)

    agg1, cnt = _sc_segment_sum(nblk, d, True)(x, src3d, dst3d, zf, iota80)
    c0 = cnt[:HR].reshape(HR * 128)[:n].reshape(n, 1)
    c1 = cnt[HR:].reshape(HR * 128)[:n].reshape(n, 1)
    h = _make_combine(n, d, True)(agg1[:n], agg1[NPAD:NPAD + n], c0, c1, x,
                                  W1l, b1.reshape(1, d), W1r)
    agg2 = _sc_segment_sum(nblk, d, False)(h, src3d, dst3d, zf, iota80)[0]
    return _make_combine(n, d, False)(agg2[:n], agg2[NPAD:NPAD + n], c0, c1, h,
                                      W2l, b2.reshape(1, d), W2r)


# speedup vs baseline: 1.5218x; 1.5218x over previous
import functools

import jax
import jax.numpy as jnp
from jax import lax
from jax.experimental import pallas as pl
from jax.experimental.pallas import tpu as pltpu
from jax.experimental.pallas import tpu_sc as plsc

NC = 2
NS = 16
NW = NC * NS
NPAD = 10112
EBLK = 512
HR = 80


def _sc_segment_sum(nblk, feat, with_cnt):
    rows_per_sub = NPAD // NS
    kmax = nblk // NW
    mesh = plsc.VectorSubcoreMesh(core_axis_name="c", subcore_axis_name="s")

    out_type = [jax.ShapeDtypeStruct((NC * NPAD, feat), jnp.float32)]
    scratch = [
        pltpu.VMEM((4, 128), jnp.int32),
        pltpu.VMEM((4, 128), jnp.int32),
        pltpu.VMEM((128, feat), jnp.float32),
        pltpu.VMEM((128, feat), jnp.float32),
        pltpu.VMEM_SHARED((NPAD, feat), jnp.float32),
        pltpu.SemaphoreType.DMA,
        pltpu.SemaphoreType.DMA,
    ]
    if with_cnt:
        out_type.append(jax.ShapeDtypeStruct((NC * HR, 128), jnp.float32))
        scratch += [
            pltpu.VMEM((HR, 128), jnp.float32),
            pltpu.VMEM((HR,), jnp.int32),
            pltpu.VMEM_SHARED((HR, 128), jnp.float32),
        ]

    @functools.partial(pl.kernel, mesh=mesh, out_type=out_type,
                       scratch_types=scratch)
    def f(x_hbm, src_hbm, dst_hbm, zf_hbm, iota_hbm, *rest):
        if with_cnt:
            (agg_out, cnt_out, srcv, dstv, rows0, rows1, agg_s, sem0, sem1,
             hist, iota_v, cnt_s) = rest
        else:
            agg_out, srcv, dstv, rows0, rows1, agg_s, sem0, sem1 = rest
        rowsb = (rows0, rows1)
        semb = (sem0, sem1)
        c = lax.axis_index("c")
        s = lax.axis_index("s")
        wid = s * NC + c
        base = s * rows_per_sub

        pltpu.sync_copy(zf_hbm, agg_s.at[pl.ds(base, rows_per_sub)])
        if with_cnt:
            pltpu.sync_copy(zf_hbm.at[pl.ds(0, HR)], hist)
            pltpu.sync_copy(iota_hbm, iota_v)
            @pl.when(s == 0)
            def _():
                pltpu.sync_copy(zf_hbm.at[pl.ds(0, HR)], cnt_s)
        plsc.subcore_barrier()

        iot = lax.iota(jnp.int32, 16)

        def histo(j):
            for t in range(8):
                dv = dstv[j, pl.ds(t * 16, 16)]
                for ll in range(16):
                    d = dv[ll]
                    hi = d // 128
                    lo = d % 128
                    off = (lo // 16) * 16
                    l16 = lo % 16
                    row = hist[hi, pl.ds(off, 16)]
                    row = row + jnp.where(iot == l16, 1.0, 0.0).astype(
                        jnp.float32)
                    hist[hi, pl.ds(off, 16)] = row

        def eblk(k, carry):
            b = wid + k * NW
            pltpu.sync_copy(src_hbm.at[b], srcv)
            pltpu.sync_copy(dst_hbm.at[b], dstv)
            hs = [None] * 4
            hs[0] = pltpu.async_copy(x_hbm.at[srcv.at[0]], rowsb[0], semb[0])
            for j in range(4):
                if with_cnt:
                    histo(j)
                hs[j].wait()
                if j < 3:
                    hs[j + 1] = pltpu.async_copy(
                        x_hbm.at[srcv.at[j + 1]], rowsb[(j + 1) % 2],
                        semb[(j + 1) % 2])
                pltpu.sync_copy(rowsb[j % 2], agg_s.at[dstv.at[j]], add=True)
            return carry
        lax.fori_loop(0, kmax, eblk, 0, unroll=False)
        if with_cnt:
            pltpu.sync_copy(hist, cnt_s.at[iota_v], add=True)
        plsc.subcore_barrier()

        out_base = c * NPAD + base
        pltpu.sync_copy(agg_s.at[pl.ds(base, rows_per_sub)],
                        agg_out.at[pl.ds(out_base, rows_per_sub)])
        if with_cnt:
            @pl.when(s == 0)
            def _():
                pltpu.sync_copy(cnt_s, cnt_out.at[pl.ds(c * HR, HR)])

    return f


def _make_combine(n_nodes, feat, relu):
    blk = 2000
    g = n_nodes // blk

    def body(a0, a1, c0, c1, xb, wl, bb, wr, ob):
        agg = a0[...] + a1[...]
        cnt = c0[...] + c1[...]
        mean = agg / jnp.maximum(cnt, 1.0)
        acc = lax.dot_general(mean, wl[...], (((1,), (1,)), ((), ())),
                              preferred_element_type=jnp.float32)
        acc = acc + bb[...]
        acc = acc + lax.dot_general(xb[...], wr[...], (((1,), (1,)), ((), ())),
                                    preferred_element_type=jnp.float32)
        if relu:
            acc = jnp.maximum(acc, 0.0)
        ob[...] = acc

    return pl.pallas_call(
        body,
        grid=(g,),
        in_specs=[
            pl.BlockSpec((blk, feat), lambda i: (i, 0)),
            pl.BlockSpec((blk, feat), lambda i: (i, 0)),
            pl.BlockSpec((blk, 1), lambda i: (i, 0)),
            pl.BlockSpec((blk, 1), lambda i: (i, 0)),
            pl.BlockSpec((blk, feat), lambda i: (i, 0)),
            pl.BlockSpec((feat, feat), lambda i: (0, 0)),
            pl.BlockSpec((1, feat), lambda i: (0, 0)),
            pl.BlockSpec((feat, feat), lambda i: (0, 0)),
        ],
        out_specs=pl.BlockSpec((blk, feat), lambda i: (i, 0)),
        out_shape=jax.ShapeDtypeStruct((n_nodes, feat), jnp.float32),
    )


def kernel(x, edge_index, W1l, b1, W1r, W2l, b2, W2r):
    n, d = x.shape
    e = edge_index.shape[1]
    epad = -e % (NW * EBLK)
    src = jnp.concatenate([edge_index[0],
                           jnp.zeros((epad,), jnp.int32)])
    dst = jnp.concatenate([edge_index[1],
                           jnp.full((epad,), NPAD - 1, jnp.int32)])
    nblk = (e + epad) // EBLK
    src3d = src.reshape(nblk, 4, 128)
    dst3d = dst.reshape(nblk, 4, 128)
    rows_per_sub = NPAD // NS
    zf = jnp.zeros((rows_per_sub, d), jnp.float32)
    iota80 = jnp.arange(HR, dtype=jnp.int32)

    agg1, cnt = _sc_segment_sum(nblk, d, True)(x, src3d, dst3d, zf, iota80)
    c0 = cnt[:HR].reshape(HR * 128)[:n].reshape(n, 1)
    c1 = cnt[HR:].reshape(HR * 128)[:n].reshape(n, 1)
    h = _make_combine(n, d, True)(agg1[:n], agg1[NPAD:NPAD + n], c0, c1, x,
                                  W1l, b1.reshape(1, d), W1r)
    agg2 = _sc_segment_sum(nblk, d, False)(h, src3d, dst3d, zf, iota80)[0]
    return _make_combine(n, d, False)(agg2[:n], agg2[NPAD:NPAD + n], c0, c1, h,
                                      W2l, b2.reshape(1, d), W2r)
